# baseline (speedup 1.0000x reference)
import functools
import math

import jax
import jax.numpy as jnp
from jax import lax
from jax.experimental import pallas as pl
from jax.experimental.pallas import tpu as pltpu
from jax.experimental.pallas import tpu_sc as plsc

D = 16
NC = 2
NS = 16
NW = NC * NS
CHUNK = 2048
GSUB = 128
LOGC = -0.5 * D * math.log(2.0 * math.pi)


@functools.partial(jax.jit, static_argnames=("m",))
def _log_prob_sc(z, idx2d, mu, m):
    per_w = m // NW
    n_chunks = per_w // CHUNK
    n_sub = CHUNK // GSUB
    n_groups = CHUNK // D

    mesh = plsc.VectorSubcoreMesh(core_axis_name="c", subcore_axis_name="s")

    @functools.partial(
        pl.kernel,
        out_type=jax.ShapeDtypeStruct((m,), jnp.float32),
        mesh=mesh,
        scratch_types=[
            pltpu.VMEM((n_sub, GSUB), jnp.int32),
            pltpu.VMEM((CHUNK, D), jnp.float32),
            pltpu.VMEM((CHUNK, D), jnp.float32),
            pltpu.VMEM((CHUNK,), jnp.float32),
            pltpu.SemaphoreType.DMA,
        ],
        compiler_params=pltpu.CompilerParams(
            use_tc_tiling_on_sc=False,
            needs_layout_passes=False,
        ),
    )
    def k(z_hbm, idx_hbm, mu_hbm, out_hbm, idx_v, mu_v, z_v, out_v, sem):
        wid = lax.axis_index("s") * NC + lax.axis_index("c")
        row_base = wid * per_w

        def chunk_body(ci, carry):
            cbase = row_base + ci * CHUNK
            irow = pl.multiple_of(cbase // GSUB, 8)
            pltpu.sync_copy(idx_hbm.at[pl.ds(irow, n_sub)], idx_v)
            pltpu.sync_copy(z_hbm.at[pl.ds(cbase, CHUNK)], z_v)
            copies = []
            for g in range(n_sub):
                copies.append(
                    pltpu.async_copy(
                        mu_hbm.at[idx_v.at[g]],
                        mu_v.at[pl.ds(g * GSUB, GSUB)],
                        sem,
                    )
                )
            for c in copies:
                c.wait()

            lane = lax.iota(jnp.int32, D)

            def group_body(g, inner):
                rows = g * D + lane
                acc = jnp.zeros((D,), jnp.float32)
                for d in range(D):
                    col = jnp.full((D,), d, jnp.int32)
                    zc = plsc.load_gather(z_v, [rows, col])
                    mc = plsc.load_gather(mu_v, [rows, col])
                    t = zc - mc
                    acc = acc + t * t
                out_v[pl.ds(g * D, D)] = -0.5 * acc + LOGC
                return inner

            lax.fori_loop(0, n_groups, group_body, 0)
            pltpu.sync_copy(out_v, out_hbm.at[pl.ds(cbase, CHUNK)])
            return carry

        lax.fori_loop(0, n_chunks, chunk_body, 0)

    return k(z, idx2d, mu)


def kernel(z, sensor_idx, mu):
    m = z.shape[0]
    idx2d = sensor_idx.astype(jnp.int32).reshape(m // GSUB, GSUB)
    return _log_prob_sc(z, idx2d, mu, m)


# speedup vs baseline: 2.4031x; 2.4031x over previous
import functools
import math

import jax
import jax.numpy as jnp
from jax import lax
from jax.experimental import pallas as pl
from jax.experimental.pallas import tpu as pltpu
from jax.experimental.pallas import tpu_sc as plsc

D = 16
NC = 2
NS = 16
NW = NC * NS
CHUNK = 2048
GSUB = 128
BB = CHUNK // 128
LOGC = -0.5 * D * math.log(2.0 * math.pi)


@functools.partial(jax.jit, static_argnames=("m",))
def _log_prob_sc(zq, idx2d, mu, m):
    per_w = m // NW
    n_chunks = per_w // CHUNK
    n_sub = CHUNK // GSUB
    n_groups = CHUNK // D

    mesh = plsc.VectorSubcoreMesh(core_axis_name="c", subcore_axis_name="s")

    @functools.partial(
        pl.kernel,
        out_type=jax.ShapeDtypeStruct((m,), jnp.float32),
        mesh=mesh,
        scratch_types=[
            pltpu.VMEM((n_sub, GSUB), jnp.int32),
            pltpu.VMEM((CHUNK, D), jnp.float32),
            pltpu.VMEM((2, BB, 8, 128), jnp.float32),
            pltpu.VMEM((CHUNK,), jnp.float32),
            pltpu.SemaphoreType.DMA,
        ],
        compiler_params=pltpu.CompilerParams(
            use_tc_tiling_on_sc=False,
            needs_layout_passes=False,
        ),
    )
    def k(zq_hbm, idx_hbm, mu_hbm, out_hbm, idx_v, mu_v, z_v, out_v, sem):
        wid = lax.axis_index("s") * NC + lax.axis_index("c")
        row_base = wid * per_w

        def chunk_body(ci, carry):
            cbase = row_base + ci * CHUNK
            irow = pl.multiple_of(cbase // GSUB, 8)
            bb0 = pl.multiple_of(cbase // 128, 8)
            pltpu.sync_copy(idx_hbm.at[pl.ds(irow, n_sub)], idx_v)
            copies = []
            for g in range(n_sub):
                copies.append(
                    pltpu.async_copy(
                        mu_hbm.at[idx_v.at[g]],
                        mu_v.at[pl.ds(g * GSUB, GSUB)],
                        sem,
                    )
                )
            for fg in range(2):
                copies.append(
                    pltpu.async_copy(
                        zq_hbm.at[fg, pl.ds(bb0, BB)], z_v.at[fg], sem
                    )
                )
            for c in copies:
                c.wait()

            lane = lax.iota(jnp.int32, D)

            def group_body(g, inner):
                bbl = g // 8
                b0 = (g % 8) * D
                rows = g * D + lane
                acc = []
                for d in range(D):
                    zc = z_v[d // 8, bbl, d % 8, pl.ds(b0, D)]
                    mc = plsc.load_gather(
                        mu_v, [rows, jnp.full((D,), d, jnp.int32)]
                    )
                    t = zc - mc
                    sq = t * t
                    if d < 4:
                        acc.append(sq)
                    else:
                        acc[d % 4] = acc[d % 4] + sq
                tot = (acc[0] + acc[1]) + (acc[2] + acc[3])
                out_v[pl.ds(g * D, D)] = -0.5 * tot + LOGC
                return inner

            lax.fori_loop(0, n_groups, group_body, 0)
            pltpu.sync_copy(out_v, out_hbm.at[pl.ds(cbase, CHUNK)])
            return carry

        lax.fori_loop(0, n_chunks, chunk_body, 0)

    return k(zq, idx2d, mu)


def kernel(z, sensor_idx, mu):
    m = z.shape[0]
    zq = z.reshape(m // 128, 128, 2, 8).transpose(2, 0, 3, 1)
    idx2d = sensor_idx.astype(jnp.int32).reshape(m // GSUB, GSUB)
    return _log_prob_sc(zq, idx2d, mu, m)
